# baseline (speedup 1.0000x reference)
import functools

import jax
import jax.numpy as jnp
from jax import lax
from jax.experimental import pallas as pl
from jax.experimental.pallas import tpu as pltpu
from jax.experimental.pallas import tpu_sc as plsc

_NC = 2
_NS = 16
_NW = _NC * _NS


def _make_table_transpose(D, V):
    n_tiles = V // 128
    tail = V - n_tiles * 128
    per_w = n_tiles // _NW
    extra = n_tiles - per_w * _NW
    mesh = plsc.VectorSubcoreMesh(core_axis_name="c", subcore_axis_name="s")

    @functools.partial(
        pl.kernel,
        mesh=mesh,
        out_type=jax.ShapeDtypeStruct((V * D,), jnp.float32),
        scratch_types=[
            pltpu.VMEM((D, 128), jnp.float32),
            pltpu.VMEM((D, 128), jnp.float32),
            pltpu.VMEM((128 * D,), jnp.float32),
            pltpu.VMEM((128 * D,), jnp.float32),
            pltpu.VMEM((64 * 64,), jnp.float32),
            pltpu.SemaphoreType.DMA,
            pltpu.SemaphoreType.DMA,
            pltpu.SemaphoreType.DMA,
            pltpu.SemaphoreType.DMA,
        ],
        compiler_params=pltpu.CompilerParams(
            use_tc_tiling_on_sc=True, needs_layout_passes=False),
    )
    def kt(tt_hbm, tail_hbm, out_hbm, b0, b1, w0, w1, tailv,
           gi0, gi1, go0, go1):
        bufs = (b0, b1)
        wbufs = (w0, w1)
        isems = (gi0, gi1)
        osems = (go0, go1)
        wid = lax.axis_index("s") * _NC + lax.axis_index("c")
        nt = per_w + jnp.where(wid < extra, 1, 0)
        tbase = per_w * wid + jnp.minimum(wid, extra)
        lane = lax.iota(jnp.int32, 16)
        bvecs = [(nb * 16 + lane) * D for nb in range(8)]

        def col(t):
            return (tbase + t) * 128

        def load(t, b):
            pltpu.async_copy(
                tt_hbm.at[:, pl.ds(col(t) * 1, 128)], bufs[b], isems[b])

        def wait_load(t, b):
            pltpu.make_async_copy(
                tt_hbm.at[:, pl.ds(col(t) * 1, 128)], bufs[b],
                isems[b]).wait()

        def store(t, b):
            pltpu.async_copy(
                wbufs[b], out_hbm.at[pl.ds(col(t) * D, 128 * D)], osems[b])

        def wait_store(t, b):
            pltpu.make_async_copy(
                wbufs[b], out_hbm.at[pl.ds(col(t) * D, 128 * D)],
                osems[b]).wait()

        def transpose(b, nblocks):
            buf = bufs[b]
            wbuf = wbufs[b]

            def per_nb(nb, carry):
                for d in range(D):
                    v = buf[d, pl.ds(nb * 16, 16)]
                    plsc.store_scatter(wbuf, [carry + d], v)
                return carry + 16 * D

            lax.fori_loop(0, nblocks, per_nb, bvecs[0])

        @pl.when(nt > 0)
        def _():
            load(0, 0)

            def body(g, carry):
                for b in range(2):
                    t = g * 2 + b

                    @pl.when(t < nt)
                    def _(t=t, b=b):
                        @pl.when(t + 1 < nt)
                        def _():
                            load(t + 1, 1 - b)

                        wait_load(t, b)

                        @pl.when(t >= 2)
                        def _():
                            wait_store(t - 2, b)

                        transpose(b, 8)
                        store(t, b)

                return carry

            lax.fori_loop(0, (per_w + 2) // 2, body, 0)

            @pl.when(lax.rem(nt, 2) == 0)
            def _():
                wait_store(nt - 2, 0)
                wait_store(nt - 1, 1)

            @pl.when(lax.rem(nt, 2) == 1)
            def _():
                wait_store(nt - 2, 1)
                wait_store(nt - 1, 0)

        @pl.when(wid == 0)
        def _():
            pltpu.sync_copy(tail_hbm, tailv)
            pltpu.sync_copy(
                tailv, out_hbm.at[pl.ds(n_tiles * 128 * D, tail * D)])

    return kt


def _make_gather(B, N, S, V, D, C, NB):
    NPW = N // _NW
    KPW = NPW * S
    n_h = NPW // C
    n_iters = S * n_h
    mesh = plsc.VectorSubcoreMesh(core_axis_name="c", subcore_axis_name="s")

    @functools.partial(
        pl.kernel,
        mesh=mesh,
        out_type=jax.ShapeDtypeStruct((B, D), jnp.float32),
        scratch_types=(
            [pltpu.VMEM((1024,), jnp.int32),
             pltpu.VMEM((KPW,), jnp.int32)]
            + [pltpu.VMEM((C, D), jnp.float32) for _ in range(NB)]
            + [pltpu.SemaphoreType.DMA for _ in range(2 * NB)]
        ),
        compiler_params=pltpu.CompilerParams(
            use_tc_tiling_on_sc=False, needs_layout_passes=False),
    )
    def kg(idx_hbm, table_hbm, out_hbm, idx_v, idx_s, *rest):
        bufs = rest[:NB]
        gsems = rest[NB:2 * NB]
        osems = rest[2 * NB:]
        wid = lax.axis_index("s") * _NC + lax.axis_index("c")
        kbase = wid * KPW
        nbase = wid * NPW
        lane = lax.iota(jnp.int32, 16)

        def stage_chunk(c2, carry):
            pltpu.sync_copy(
                idx_hbm.at[pl.ds(kbase + c2 * 1024, 1024)], idx_v)

            def scat(kb, carry2):
                kl = c2 * 1024 + kb * 16
                vals = idx_v[pl.ds(kb * 16, 16)]
                kvec = kl + lane
                svec = lax.rem(kvec, S)
                nvec = lax.div(kvec, S)
                plsc.store_scatter(idx_s, [svec * NPW + nvec], vals)
                return carry2

            lax.fori_loop(0, 64, scat, 0)
            return carry

        lax.fori_loop(0, KPW // 1024, stage_chunk, 0)

        def gather(it, b):
            pltpu.async_copy(
                table_hbm.at[idx_s.at[pl.ds(it * C, C)]], bufs[b], gsems[b])

        def wait_gather(it, b):
            pltpu.make_async_copy(
                table_hbm.at[idx_s.at[pl.ds(it * C, C)]], bufs[b],
                gsems[b]).wait()

        def orow(it):
            s = it // n_h
            h = it % n_h
            return s * N + nbase + h * C

        def write(it, b):
            pltpu.async_copy(
                bufs[b], out_hbm.at[pl.ds(orow(it), C)], osems[b])

        def wait_write(it, b):
            pltpu.make_async_copy(
                bufs[b], out_hbm.at[pl.ds(orow(it), C)], osems[b]).wait()

        for b in range(NB):
            gather(b, b)

        def body(g, carry):
            for b in range(NB):
                it = g * NB + b
                wait_gather(it, b)
                write(it, b)

            @pl.when(g + 1 < n_iters // NB)
            def _():
                for b in range(NB):
                    it = g * NB + b
                    wait_write(it, b)
                    gather(it + NB, b)

            return carry

        lax.fori_loop(0, n_iters // NB, body, 0)
        for b in range(NB):
            wait_write(n_iters - NB + b, b)

    return kg


def _make_permute(B, N, S, D, NCH):
    NPW = N // _NW
    n_h = NPW // NCH
    n_iters = S * n_h
    mesh = plsc.VectorSubcoreMesh(core_axis_name="c", subcore_axis_name="s")

    @functools.partial(
        pl.kernel,
        mesh=mesh,
        out_type=jax.ShapeDtypeStruct((S, D, N), jnp.float32),
        scratch_types=[
            pltpu.VMEM((NCH * D,), jnp.float32),
            pltpu.VMEM((NCH * D,), jnp.float32),
            pltpu.VMEM((D, NCH), jnp.float32),
            pltpu.VMEM((D, NCH), jnp.float32),
            pltpu.SemaphoreType.DMA,
            pltpu.SemaphoreType.DMA,
            pltpu.SemaphoreType.DMA,
            pltpu.SemaphoreType.DMA,
        ],
        compiler_params=pltpu.CompilerParams(
            use_tc_tiling_on_sc=True, needs_layout_passes=False),
    )
    def kp(in_hbm, out_hbm, c0, c1, w0, w1, gi0, gi1, go0, go1):
        cbufs = (c0, c1)
        wbufs = (w0, w1)
        isems = (gi0, gi1)
        osems = (go0, go1)
        wid = lax.axis_index("s") * _NC + lax.axis_index("c")
        nbase = wid * NPW
        lane = lax.iota(jnp.int32, 16)
        dvecs = [jnp.int32(d0) + lane for d0 in range(0, D, 16)]

        def ibase(it):
            s = it // n_h
            h = it % n_h
            return (s * N + nbase + h * NCH) * D

        def load(it, b):
            pltpu.async_copy(
                in_hbm.at[pl.ds(ibase(it), NCH * D)], cbufs[b], isems[b])

        def wait_load(it, b):
            pltpu.make_async_copy(
                in_hbm.at[pl.ds(ibase(it), NCH * D)], cbufs[b],
                isems[b]).wait()

        def store(it, b):
            s = it // n_h
            h = it % n_h
            pltpu.async_copy(
                wbufs[b], out_hbm.at[s, :, pl.ds(nbase + h * NCH, NCH)],
                osems[b])

        def wait_store(it, b):
            s = it // n_h
            h = it % n_h
            pltpu.make_async_copy(
                wbufs[b], out_hbm.at[s, :, pl.ds(nbase + h * NCH, NCH)],
                osems[b]).wait()

        def shuffle(b):
            cbuf = cbufs[b]
            wbuf = wbufs[b]

            def per_n(g, nvec):
                for u in range(4):
                    n = g * 4 + u
                    nv = nvec + u
                    for i in range(D // 16):
                        v = cbuf[pl.ds(n * D + i * 16, 16)]
                        plsc.store_scatter(wbuf, [dvecs[i], nv], v)
                return nvec + 4

            lax.fori_loop(0, NCH // 4, per_n, jnp.zeros((16,), jnp.int32))

        load(0, 0)

        def body(g, carry):
            for b in range(2):
                it = g * 2 + b

                @pl.when(it + 1 < n_iters)
                def _(it=it, b=b):
                    load(it + 1, 1 - b)

                wait_load(it, b)

                @pl.when(it >= 2)
                def _(it=it, b=b):
                    wait_store(it - 2, b)

                shuffle(b)
                store(it, b)
            return carry

        lax.fori_loop(0, n_iters // 2, body, 0)
        wait_store(n_iters - 2, 0)
        wait_store(n_iters - 1, 1)

    return kp


def kernel(input, table):
    N, S = input.shape
    V, D = table.shape
    B = N * S
    tt = jnp.transpose(table)
    tail_1d = table[V - 64:, :].reshape(-1)
    t1d = _make_table_transpose(D, V)(tt, tail_1d)
    tableu = t1d.reshape(V, D)
    idxf = input.reshape(-1)
    gsm = _make_gather(B, N, S, V, D, 256, 4)(idxf, tableu)
    out_t = _make_permute(B, N, S, D, 256)(gsm.reshape(-1))
    return jnp.transpose(out_t, (2, 0, 1))


# speedup vs baseline: 1.0013x; 1.0013x over previous
import functools

import jax
import jax.numpy as jnp
from jax import lax
from jax.experimental import pallas as pl
from jax.experimental.pallas import tpu as pltpu
from jax.experimental.pallas import tpu_sc as plsc

_NC = 2
_NS = 16
_NW = _NC * _NS


def _make_table_transpose(D, V):
    n_tiles = V // 128
    tail = V - n_tiles * 128
    per_w = n_tiles // _NW
    extra = n_tiles - per_w * _NW
    mesh = plsc.VectorSubcoreMesh(core_axis_name="c", subcore_axis_name="s")

    @functools.partial(
        pl.kernel,
        mesh=mesh,
        out_type=jax.ShapeDtypeStruct((V * D,), jnp.float32),
        scratch_types=[
            pltpu.VMEM((D, 128), jnp.float32),
            pltpu.VMEM((D, 128), jnp.float32),
            pltpu.VMEM((128 * D,), jnp.float32),
            pltpu.VMEM((128 * D,), jnp.float32),
            pltpu.VMEM((64 * 64,), jnp.float32),
            pltpu.SemaphoreType.DMA,
            pltpu.SemaphoreType.DMA,
            pltpu.SemaphoreType.DMA,
            pltpu.SemaphoreType.DMA,
        ],
        compiler_params=pltpu.CompilerParams(
            use_tc_tiling_on_sc=True, needs_layout_passes=False,
            disable_bounds_checks=True),
    )
    def kt(tt_hbm, tail_hbm, out_hbm, b0, b1, w0, w1, tailv,
           gi0, gi1, go0, go1):
        bufs = (b0, b1)
        wbufs = (w0, w1)
        isems = (gi0, gi1)
        osems = (go0, go1)
        wid = lax.axis_index("s") * _NC + lax.axis_index("c")
        nt = per_w + jnp.where(wid < extra, 1, 0)
        tbase = per_w * wid + jnp.minimum(wid, extra)
        lane = lax.iota(jnp.int32, 16)
        bvecs = [(nb * 16 + lane) * D for nb in range(8)]

        def col(t):
            return (tbase + t) * 128

        def load(t, b):
            pltpu.async_copy(
                tt_hbm.at[:, pl.ds(col(t) * 1, 128)], bufs[b], isems[b])

        def wait_load(t, b):
            pltpu.make_async_copy(
                tt_hbm.at[:, pl.ds(col(t) * 1, 128)], bufs[b],
                isems[b]).wait()

        def store(t, b):
            pltpu.async_copy(
                wbufs[b], out_hbm.at[pl.ds(col(t) * D, 128 * D)], osems[b])

        def wait_store(t, b):
            pltpu.make_async_copy(
                wbufs[b], out_hbm.at[pl.ds(col(t) * D, 128 * D)],
                osems[b]).wait()

        def transpose(b, nblocks):
            buf = bufs[b]
            wbuf = wbufs[b]

            def per_nb(nb, carry):
                for d in range(D):
                    v = buf[d, pl.ds(nb * 16, 16)]
                    plsc.store_scatter(wbuf, [carry + d], v)
                return carry + 16 * D

            lax.fori_loop(0, nblocks, per_nb, bvecs[0])

        @pl.when(nt > 0)
        def _():
            load(0, 0)

            def body(g, carry):
                for b in range(2):
                    t = g * 2 + b

                    @pl.when(t < nt)
                    def _(t=t, b=b):
                        @pl.when(t + 1 < nt)
                        def _():
                            load(t + 1, 1 - b)

                        wait_load(t, b)

                        @pl.when(t >= 2)
                        def _():
                            wait_store(t - 2, b)

                        transpose(b, 8)
                        store(t, b)

                return carry

            lax.fori_loop(0, (per_w + 2) // 2, body, 0)

            @pl.when(lax.rem(nt, 2) == 0)
            def _():
                wait_store(nt - 2, 0)
                wait_store(nt - 1, 1)

            @pl.when(lax.rem(nt, 2) == 1)
            def _():
                wait_store(nt - 2, 1)
                wait_store(nt - 1, 0)

        @pl.when(wid == 0)
        def _():
            pltpu.sync_copy(tail_hbm, tailv)
            pltpu.sync_copy(
                tailv, out_hbm.at[pl.ds(n_tiles * 128 * D, tail * D)])

    return kt


def _make_gather(B, N, S, V, D, C, NB):
    NPW = N // _NW
    KPW = NPW * S
    n_h = NPW // C
    n_iters = S * n_h
    mesh = plsc.VectorSubcoreMesh(core_axis_name="c", subcore_axis_name="s")

    @functools.partial(
        pl.kernel,
        mesh=mesh,
        out_type=jax.ShapeDtypeStruct((B, D), jnp.float32),
        scratch_types=(
            [pltpu.VMEM((1024,), jnp.int32),
             pltpu.VMEM((KPW,), jnp.int32)]
            + [pltpu.VMEM((C, D), jnp.float32) for _ in range(NB)]
            + [pltpu.SemaphoreType.DMA for _ in range(2 * NB)]
        ),
        compiler_params=pltpu.CompilerParams(
            use_tc_tiling_on_sc=False, needs_layout_passes=False,
            disable_bounds_checks=True),
    )
    def kg(idx_hbm, table_hbm, out_hbm, idx_v, idx_s, *rest):
        bufs = rest[:NB]
        gsems = rest[NB:2 * NB]
        osems = rest[2 * NB:]
        wid = lax.axis_index("s") * _NC + lax.axis_index("c")
        kbase = wid * KPW
        nbase = wid * NPW
        lane = lax.iota(jnp.int32, 16)

        def stage_chunk(c2, carry):
            pltpu.sync_copy(
                idx_hbm.at[pl.ds(kbase + c2 * 1024, 1024)], idx_v)

            def scat(kb, carry2):
                kl = c2 * 1024 + kb * 16
                vals = idx_v[pl.ds(kb * 16, 16)]
                kvec = kl + lane
                svec = lax.rem(kvec, S)
                nvec = lax.div(kvec, S)
                plsc.store_scatter(idx_s, [svec * NPW + nvec], vals)
                return carry2

            lax.fori_loop(0, 64, scat, 0)
            return carry

        lax.fori_loop(0, KPW // 1024, stage_chunk, 0)

        def gather(it, b):
            pltpu.async_copy(
                table_hbm.at[idx_s.at[pl.ds(it * C, C)]], bufs[b], gsems[b])

        def wait_gather(it, b):
            pltpu.make_async_copy(
                table_hbm.at[idx_s.at[pl.ds(it * C, C)]], bufs[b],
                gsems[b]).wait()

        def orow(it):
            s = it // n_h
            h = it % n_h
            return s * N + nbase + h * C

        def write(it, b):
            pltpu.async_copy(
                bufs[b], out_hbm.at[pl.ds(orow(it), C)], osems[b])

        def wait_write(it, b):
            pltpu.make_async_copy(
                bufs[b], out_hbm.at[pl.ds(orow(it), C)], osems[b]).wait()

        for b in range(NB):
            gather(b, b)

        def body(g, carry):
            for b in range(NB):
                it = g * NB + b
                wait_gather(it, b)
                write(it, b)

            @pl.when(g + 1 < n_iters // NB)
            def _():
                for b in range(NB):
                    it = g * NB + b
                    wait_write(it, b)
                    gather(it + NB, b)

            return carry

        lax.fori_loop(0, n_iters // NB, body, 0)
        for b in range(NB):
            wait_write(n_iters - NB + b, b)

    return kg


def _make_permute(B, N, S, D, NCH):
    NPW = N // _NW
    n_h = NPW // NCH
    n_iters = S * n_h
    mesh = plsc.VectorSubcoreMesh(core_axis_name="c", subcore_axis_name="s")

    @functools.partial(
        pl.kernel,
        mesh=mesh,
        out_type=jax.ShapeDtypeStruct((S, D, N), jnp.float32),
        scratch_types=[
            pltpu.VMEM((NCH * D,), jnp.float32),
            pltpu.VMEM((NCH * D,), jnp.float32),
            pltpu.VMEM((D, NCH), jnp.float32),
            pltpu.VMEM((D, NCH), jnp.float32),
            pltpu.SemaphoreType.DMA,
            pltpu.SemaphoreType.DMA,
            pltpu.SemaphoreType.DMA,
            pltpu.SemaphoreType.DMA,
        ],
        compiler_params=pltpu.CompilerParams(
            use_tc_tiling_on_sc=True, needs_layout_passes=False,
            disable_bounds_checks=True),
    )
    def kp(in_hbm, out_hbm, c0, c1, w0, w1, gi0, gi1, go0, go1):
        cbufs = (c0, c1)
        wbufs = (w0, w1)
        isems = (gi0, gi1)
        osems = (go0, go1)
        wid = lax.axis_index("s") * _NC + lax.axis_index("c")
        nbase = wid * NPW
        lane = lax.iota(jnp.int32, 16)
        dvecs = [jnp.int32(d0) + lane for d0 in range(0, D, 16)]

        def ibase(it):
            s = it // n_h
            h = it % n_h
            return (s * N + nbase + h * NCH) * D

        def load(it, b):
            pltpu.async_copy(
                in_hbm.at[pl.ds(ibase(it), NCH * D)], cbufs[b], isems[b])

        def wait_load(it, b):
            pltpu.make_async_copy(
                in_hbm.at[pl.ds(ibase(it), NCH * D)], cbufs[b],
                isems[b]).wait()

        def store(it, b):
            s = it // n_h
            h = it % n_h
            pltpu.async_copy(
                wbufs[b], out_hbm.at[s, :, pl.ds(nbase + h * NCH, NCH)],
                osems[b])

        def wait_store(it, b):
            s = it // n_h
            h = it % n_h
            pltpu.make_async_copy(
                wbufs[b], out_hbm.at[s, :, pl.ds(nbase + h * NCH, NCH)],
                osems[b]).wait()

        def shuffle(b):
            cbuf = cbufs[b]
            wbuf = wbufs[b]

            def per_n(g, nvec):
                for u in range(4):
                    n = g * 4 + u
                    nv = nvec + u
                    for i in range(D // 16):
                        v = cbuf[pl.ds(n * D + i * 16, 16)]
                        plsc.store_scatter(wbuf, [dvecs[i], nv], v)
                return nvec + 4

            lax.fori_loop(0, NCH // 4, per_n, jnp.zeros((16,), jnp.int32))

        load(0, 0)

        def body(g, carry):
            for b in range(2):
                it = g * 2 + b

                @pl.when(it + 1 < n_iters)
                def _(it=it, b=b):
                    load(it + 1, 1 - b)

                wait_load(it, b)

                @pl.when(it >= 2)
                def _(it=it, b=b):
                    wait_store(it - 2, b)

                shuffle(b)
                store(it, b)
            return carry

        lax.fori_loop(0, n_iters // 2, body, 0)
        wait_store(n_iters - 2, 0)
        wait_store(n_iters - 1, 1)

    return kp


def kernel(input, table):
    N, S = input.shape
    V, D = table.shape
    B = N * S
    tt = jnp.transpose(table)
    tail_1d = table[V - 64:, :].reshape(-1)
    t1d = _make_table_transpose(D, V)(tt, tail_1d)
    tableu = t1d.reshape(V, D)
    idxf = input.reshape(-1)
    gsm = _make_gather(B, N, S, V, D, 256, 4)(idxf, tableu)
    out_t = _make_permute(B, N, S, D, 256)(gsm.reshape(-1))
    return jnp.transpose(out_t, (2, 0, 1))


# speedup vs baseline: 1.2227x; 1.2211x over previous
import functools

import jax
import jax.numpy as jnp
from jax import lax
from jax.experimental import pallas as pl
from jax.experimental.pallas import tpu as pltpu
from jax.experimental.pallas import tpu_sc as plsc

_NC = 2
_NS = 16
_NW = _NC * _NS


def _make_table_transpose(D, V):
    n_tiles = V // 128
    tail = V - n_tiles * 128
    per_w = n_tiles // _NW
    extra = n_tiles - per_w * _NW
    mesh = plsc.VectorSubcoreMesh(core_axis_name="c", subcore_axis_name="s")

    @functools.partial(
        pl.kernel,
        mesh=mesh,
        out_type=jax.ShapeDtypeStruct((V * D,), jnp.float32),
        scratch_types=[
            pltpu.VMEM((D, 128), jnp.float32),
            pltpu.VMEM((D, 128), jnp.float32),
            pltpu.VMEM((128 * D,), jnp.float32),
            pltpu.VMEM((128 * D,), jnp.float32),
            pltpu.VMEM((64 * 64,), jnp.float32),
            pltpu.SemaphoreType.DMA,
            pltpu.SemaphoreType.DMA,
            pltpu.SemaphoreType.DMA,
            pltpu.SemaphoreType.DMA,
        ],
        compiler_params=pltpu.CompilerParams(
            use_tc_tiling_on_sc=True, needs_layout_passes=False,
            disable_bounds_checks=True),
    )
    def kt(tt_hbm, tail_hbm, out_hbm, b0, b1, w0, w1, tailv,
           gi0, gi1, go0, go1):
        bufs = (b0, b1)
        wbufs = (w0, w1)
        isems = (gi0, gi1)
        osems = (go0, go1)
        wid = lax.axis_index("s") * _NC + lax.axis_index("c")
        nt = per_w + jnp.where(wid < extra, 1, 0)
        tbase = per_w * wid + jnp.minimum(wid, extra)
        lane = lax.iota(jnp.int32, 16)
        bvecs = [(nb * 16 + lane) * D for nb in range(8)]

        def col(t):
            return (tbase + t) * 128

        def load(t, b):
            pltpu.async_copy(
                tt_hbm.at[:, pl.ds(col(t) * 1, 128)], bufs[b], isems[b])

        def wait_load(t, b):
            pltpu.make_async_copy(
                tt_hbm.at[:, pl.ds(col(t) * 1, 128)], bufs[b],
                isems[b]).wait()

        def store(t, b):
            pltpu.async_copy(
                wbufs[b], out_hbm.at[pl.ds(col(t) * D, 128 * D)], osems[b])

        def wait_store(t, b):
            pltpu.make_async_copy(
                wbufs[b], out_hbm.at[pl.ds(col(t) * D, 128 * D)],
                osems[b]).wait()

        def transpose(b, nblocks):
            buf = bufs[b]
            wbuf = wbufs[b]

            @plsc.parallel_loop(0, nblocks, carry=bvecs[0])
            def per_nb(nb, carry):
                for d in range(D):
                    v = buf[d, pl.ds(nb * 16, 16)]
                    plsc.store_scatter(wbuf, [carry + d], v)
                return carry + 16 * D

        @pl.when(nt > 0)
        def _():
            load(0, 0)

            def body(g, carry):
                for b in range(2):
                    t = g * 2 + b

                    @pl.when(t < nt)
                    def _(t=t, b=b):
                        @pl.when(t + 1 < nt)
                        def _():
                            load(t + 1, 1 - b)

                        wait_load(t, b)

                        @pl.when(t >= 2)
                        def _():
                            wait_store(t - 2, b)

                        transpose(b, 8)
                        store(t, b)

                return carry

            lax.fori_loop(0, (per_w + 2) // 2, body, 0)

            @pl.when(lax.rem(nt, 2) == 0)
            def _():
                wait_store(nt - 2, 0)
                wait_store(nt - 1, 1)

            @pl.when(lax.rem(nt, 2) == 1)
            def _():
                wait_store(nt - 2, 1)
                wait_store(nt - 1, 0)

        @pl.when(wid == 0)
        def _():
            pltpu.sync_copy(tail_hbm, tailv)
            pltpu.sync_copy(
                tailv, out_hbm.at[pl.ds(n_tiles * 128 * D, tail * D)])

    return kt


def _make_gather(B, N, S, V, D, C, NB):
    NPW = N // _NW
    KPW = NPW * S
    n_h = NPW // C
    n_iters = S * n_h
    mesh = plsc.VectorSubcoreMesh(core_axis_name="c", subcore_axis_name="s")

    @functools.partial(
        pl.kernel,
        mesh=mesh,
        out_type=jax.ShapeDtypeStruct((B, D), jnp.float32),
        scratch_types=(
            [pltpu.VMEM((1024,), jnp.int32),
             pltpu.VMEM((KPW,), jnp.int32)]
            + [pltpu.VMEM((C, D), jnp.float32) for _ in range(NB)]
            + [pltpu.SemaphoreType.DMA for _ in range(2 * NB)]
        ),
        compiler_params=pltpu.CompilerParams(
            use_tc_tiling_on_sc=False, needs_layout_passes=False,
            disable_bounds_checks=True),
    )
    def kg(idx_hbm, table_hbm, out_hbm, idx_v, idx_s, *rest):
        bufs = rest[:NB]
        gsems = rest[NB:2 * NB]
        osems = rest[2 * NB:]
        wid = lax.axis_index("s") * _NC + lax.axis_index("c")
        kbase = wid * KPW
        nbase = wid * NPW
        lane = lax.iota(jnp.int32, 16)

        def stage_chunk(c2, carry):
            pltpu.sync_copy(
                idx_hbm.at[pl.ds(kbase + c2 * 1024, 1024)], idx_v)

            def scat(kb, carry2):
                kl = c2 * 1024 + kb * 16
                vals = idx_v[pl.ds(kb * 16, 16)]
                kvec = kl + lane
                svec = lax.rem(kvec, S)
                nvec = lax.div(kvec, S)
                plsc.store_scatter(idx_s, [svec * NPW + nvec], vals)
                return carry2

            lax.fori_loop(0, 64, scat, 0)
            return carry

        lax.fori_loop(0, KPW // 1024, stage_chunk, 0)

        def gather(it, b):
            pltpu.async_copy(
                table_hbm.at[idx_s.at[pl.ds(it * C, C)]], bufs[b], gsems[b])

        def wait_gather(it, b):
            pltpu.make_async_copy(
                table_hbm.at[idx_s.at[pl.ds(it * C, C)]], bufs[b],
                gsems[b]).wait()

        def orow(it):
            s = it // n_h
            h = it % n_h
            return s * N + nbase + h * C

        def write(it, b):
            pltpu.async_copy(
                bufs[b], out_hbm.at[pl.ds(orow(it), C)], osems[b])

        def wait_write(it, b):
            pltpu.make_async_copy(
                bufs[b], out_hbm.at[pl.ds(orow(it), C)], osems[b]).wait()

        for b in range(NB):
            gather(b, b)

        def body(g, carry):
            for b in range(NB):
                it = g * NB + b
                wait_gather(it, b)
                write(it, b)

            @pl.when(g + 1 < n_iters // NB)
            def _():
                for b in range(NB):
                    it = g * NB + b
                    wait_write(it, b)
                    gather(it + NB, b)

            return carry

        lax.fori_loop(0, n_iters // NB, body, 0)
        for b in range(NB):
            wait_write(n_iters - NB + b, b)

    return kg


def _make_permute(B, N, S, D, NCH):
    NPW = N // _NW
    n_h = NPW // NCH
    n_iters = S * n_h
    mesh = plsc.VectorSubcoreMesh(core_axis_name="c", subcore_axis_name="s")

    @functools.partial(
        pl.kernel,
        mesh=mesh,
        out_type=jax.ShapeDtypeStruct((S, D, N), jnp.float32),
        scratch_types=[
            pltpu.VMEM((NCH * D,), jnp.float32),
            pltpu.VMEM((NCH * D,), jnp.float32),
            pltpu.VMEM((D, NCH), jnp.float32),
            pltpu.VMEM((D, NCH), jnp.float32),
            pltpu.SemaphoreType.DMA,
            pltpu.SemaphoreType.DMA,
            pltpu.SemaphoreType.DMA,
            pltpu.SemaphoreType.DMA,
        ],
        compiler_params=pltpu.CompilerParams(
            use_tc_tiling_on_sc=True, needs_layout_passes=False,
            disable_bounds_checks=True),
    )
    def kp(in_hbm, out_hbm, c0, c1, w0, w1, gi0, gi1, go0, go1):
        cbufs = (c0, c1)
        wbufs = (w0, w1)
        isems = (gi0, gi1)
        osems = (go0, go1)
        wid = lax.axis_index("s") * _NC + lax.axis_index("c")
        nbase = wid * NPW
        lane = lax.iota(jnp.int32, 16)
        dvecs = [jnp.int32(d0) + lane for d0 in range(0, D, 16)]

        def ibase(it):
            s = it // n_h
            h = it % n_h
            return (s * N + nbase + h * NCH) * D

        def load(it, b):
            pltpu.async_copy(
                in_hbm.at[pl.ds(ibase(it), NCH * D)], cbufs[b], isems[b])

        def wait_load(it, b):
            pltpu.make_async_copy(
                in_hbm.at[pl.ds(ibase(it), NCH * D)], cbufs[b],
                isems[b]).wait()

        def store(it, b):
            s = it // n_h
            h = it % n_h
            pltpu.async_copy(
                wbufs[b], out_hbm.at[s, :, pl.ds(nbase + h * NCH, NCH)],
                osems[b])

        def wait_store(it, b):
            s = it // n_h
            h = it % n_h
            pltpu.make_async_copy(
                wbufs[b], out_hbm.at[s, :, pl.ds(nbase + h * NCH, NCH)],
                osems[b]).wait()

        def shuffle(b):
            cbuf = cbufs[b]
            wbuf = wbufs[b]

            @plsc.parallel_loop(0, NCH // 4, carry=jnp.zeros((16,), jnp.int32))
            def per_n(g, nvec):
                for u in range(4):
                    n = g * 4 + u
                    nv = nvec + u
                    for i in range(D // 16):
                        v = cbuf[pl.ds(n * D + i * 16, 16)]
                        plsc.store_scatter(wbuf, [dvecs[i], nv], v)
                return nvec + 4

        load(0, 0)

        def body(g, carry):
            for b in range(2):
                it = g * 2 + b

                @pl.when(it + 1 < n_iters)
                def _(it=it, b=b):
                    load(it + 1, 1 - b)

                wait_load(it, b)

                @pl.when(it >= 2)
                def _(it=it, b=b):
                    wait_store(it - 2, b)

                shuffle(b)
                store(it, b)
            return carry

        lax.fori_loop(0, n_iters // 2, body, 0)
        wait_store(n_iters - 2, 0)
        wait_store(n_iters - 1, 1)

    return kp


def kernel(input, table):
    N, S = input.shape
    V, D = table.shape
    B = N * S
    tt = jnp.transpose(table)
    tail_1d = table[V - 64:, :].reshape(-1)
    t1d = _make_table_transpose(D, V)(tt, tail_1d)
    tableu = t1d.reshape(V, D)
    idxf = input.reshape(-1)
    gsm = _make_gather(B, N, S, V, D, 256, 4)(idxf, tableu)
    out_t = _make_permute(B, N, S, D, 256)(gsm.reshape(-1))
    return jnp.transpose(out_t, (2, 0, 1))


# speedup vs baseline: 1.9039x; 1.5571x over previous
import functools

import jax
import jax.numpy as jnp
from jax import lax
from jax.experimental import pallas as pl
from jax.experimental.pallas import tpu as pltpu
from jax.experimental.pallas import tpu_sc as plsc

_NC = 2
_NS = 16
_NW = _NC * _NS


def _make_gather(N, S, V, D, R, NB):
    assert N % (_NW * R * NB) == 0
    rows_per_w = N // _NW
    n_chunks = rows_per_w // R
    n_groups = n_chunks // NB
    mesh = plsc.VectorSubcoreMesh(core_axis_name="c", subcore_axis_name="s")

    scratch = (
        [pltpu.VMEM((rows_per_w, S), jnp.int32)]
        + [pltpu.VMEM((R, S, D), jnp.float32) for _ in range(NB)]
        + [pltpu.SemaphoreType.DMA for _ in range(2 * NB)]
    )

    @functools.partial(
        pl.kernel,
        mesh=mesh,
        out_type=jax.ShapeDtypeStruct((N, S, D), jnp.float32),
        scratch_types=scratch,
        compiler_params=pltpu.CompilerParams(use_tc_tiling_on_sc=False),
    )
    def gather_kernel(idx_hbm, table_hbm, out_hbm, idx_v, *rest):
        bufs = rest[:NB]
        gsems = rest[NB:2 * NB]
        osems = rest[2 * NB:]
        wid = lax.axis_index("s") * _NC + lax.axis_index("c")
        row_base = wid * rows_per_w
        pltpu.sync_copy(idx_hbm.at[pl.ds(row_base, rows_per_w)], idx_v)

        def gather(j, b):
            for r in range(R):
                pltpu.async_copy(
                    table_hbm.at[idx_v.at[j * R + r]], bufs[b].at[r],
                    gsems[b])

        def wait_gather(j, b):
            for r in range(R):
                pltpu.make_async_copy(
                    table_hbm.at[idx_v.at[j * R + r]], bufs[b].at[r],
                    gsems[b]).wait()

        def write(j, b):
            pltpu.async_copy(
                bufs[b], out_hbm.at[pl.ds(row_base + j * R, R)], osems[b])

        def wait_write(j, b):
            pltpu.make_async_copy(
                bufs[b], out_hbm.at[pl.ds(row_base + j * R, R)],
                osems[b]).wait()

        for b in range(NB):
            gather(b, b)

        def body(g, carry):
            for b in range(NB):
                wait_gather(g * NB + b, b)
                write(g * NB + b, b)

            @pl.when(g + 1 < n_groups)
            def _():
                for b in range(NB):
                    wait_write(g * NB + b, b)
                    gather((g + 1) * NB + b, b)

            return carry

        lax.fori_loop(0, n_groups, body, 0)
        for b in range(NB):
            wait_write((n_groups - 1) * NB + b, b)

    return gather_kernel


def kernel(input, table):
    N, S = input.shape
    V, D = table.shape
    return _make_gather(N, S, V, D, 16, 4)(input, table)
